# baseline (speedup 1.0000x reference)
import functools
import math

import jax
import jax.numpy as jnp
from jax import lax
from jax.experimental import pallas as pl
from jax.experimental.pallas import tpu as pltpu
from jax.experimental.pallas import tpu_sc as plsc

N_E = 8192
E_DIM = 256
BETA = 0.25
TOK = 9216
TB = 256
NB = TOK // TB
NW = 32
BPW = TOK // NW


def _norm_kernel(w_ref, cb_ref, cbn_ref):
    w = w_ref[...]
    n1 = jnp.sqrt(jnp.sum(w * w, axis=1, keepdims=True))
    cb = w / jnp.maximum(n1, 1e-12)
    n2 = jnp.sqrt(jnp.sum(cb * cb, axis=1, keepdims=True))
    cb_ref[...] = cb
    cbn_ref[...] = cb / jnp.maximum(n2, 1e-12)


def _vq_kernel(z_ref, cbn_ref, idx_ref, p_ref, rm_ref, d0_ref, d1_ref):
    i = pl.program_id(0)

    @pl.when(i == 0)
    def _init():
        p_ref[...] = jnp.zeros_like(p_ref)
        rm_ref[...] = jnp.zeros_like(rm_ref)
        d1_ref[...] = jnp.zeros_like(d1_ref)

    def _mm(dst_ref):
        z = z_ref[...]
        nz = jnp.sqrt(jnp.sum(z * z, axis=1, keepdims=True))
        zn = z / jnp.maximum(nz, 1e-12)
        dst_ref[...] = jax.lax.dot_general(
            zn, cbn_ref[...], (((1,), (1,)), ((), ())),
            preferred_element_type=jnp.float32)

    def _post(src_ref):
        d = src_ref[...]
        rmax = jnp.max(d, axis=1, keepdims=True)
        iota = jax.lax.broadcasted_iota(jnp.int32, d.shape, 1)
        idx = jnp.min(jnp.where(d == rmax, iota, N_E), axis=1, keepdims=True)
        idx_ref[...] = idx

        s = jnp.exp(d - rmax)
        rs = jnp.sum(s, axis=1, keepdims=True)
        pb = jnp.sum(s * (1.0 / rs), axis=0, keepdims=True)
        valid = jnp.where(i > 0, 1.0, 0.0).astype(jnp.float32)
        p_ref[...] += pb * valid
        rm_ref[...] += rmax

    @pl.when(i % 2 == 0)
    def _even():
        _mm(d0_ref)
        _post(d1_ref)

    @pl.when(i % 2 == 1)
    def _odd():
        _mm(d1_ref)
        _post(d0_ref)


def _gather_body(cb_hbm, idx_hbm, zero_hbm, zq_hbm, hist_hbm,
                 idx_v, rows_v, ones_v, shared, sem):
    c = lax.axis_index("c")
    s = lax.axis_index("s")
    wid = s * 2 + c
    base = wid * BPW
    pltpu.sync_copy(idx_hbm.at[pl.ds(base, BPW)], idx_v)
    pltpu.async_copy(cb_hbm.at[idx_v], rows_v, sem).wait()
    pltpu.sync_copy(rows_v, zq_hbm.at[pl.ds(base, BPW)])

    def _fill(k, _):
        ones_v[pl.ds(k * 16, 16)] = jnp.ones((16,), jnp.float32)
        return 0
    lax.fori_loop(0, BPW // 16, _fill, 0)

    @pl.when(s == 0)
    def _zero():
        pltpu.sync_copy(zero_hbm, shared)
    plsc.subcore_barrier()
    pltpu.sync_copy(ones_v, shared.at[idx_v], add=True)
    plsc.subcore_barrier()

    @pl.when(s == 0)
    def _write():
        pltpu.sync_copy(shared, hist_hbm.at[c])


def _sc_gather_hist(cb, idx, zero):
    mesh = plsc.VectorSubcoreMesh(core_axis_name="c", subcore_axis_name="s")
    return pl.kernel(
        _gather_body,
        out_type=[
            jax.ShapeDtypeStruct((TOK, E_DIM), jnp.float32),
            jax.ShapeDtypeStruct((2, N_E), jnp.float32),
        ],
        mesh=mesh,
        scratch_types=[
            pltpu.VMEM((BPW,), jnp.int32),
            pltpu.VMEM((BPW, E_DIM), jnp.float32),
            pltpu.VMEM((BPW,), jnp.float32),
            pltpu.VMEM_SHARED((N_E,), jnp.float32),
            pltpu.SemaphoreType.DMA,
        ],
    )(cb, idx, zero)


@functools.partial(jax.jit, static_argnames=())
def kernel(z, W):
    z_flat = z.reshape(-1, E_DIM)

    cb, cbn = pl.pallas_call(
        _norm_kernel,
        out_shape=[
            jax.ShapeDtypeStruct((N_E, E_DIM), jnp.float32),
            jax.ShapeDtypeStruct((N_E, E_DIM), jnp.float32),
        ],
        in_specs=[pl.BlockSpec((N_E, E_DIM), lambda: (0, 0))],
        out_specs=[
            pl.BlockSpec((N_E, E_DIM), lambda: (0, 0)),
            pl.BlockSpec((N_E, E_DIM), lambda: (0, 0)),
        ],
    )(W)

    idx, p_sum, rm_acc = pl.pallas_call(
        _vq_kernel,
        grid=(NB + 1,),
        out_shape=[
            jax.ShapeDtypeStruct((TOK, 1), jnp.int32),
            jax.ShapeDtypeStruct((1, N_E), jnp.float32),
            jax.ShapeDtypeStruct((TB, 1), jnp.float32),
        ],
        in_specs=[
            pl.BlockSpec((TB, E_DIM), lambda i: (i % NB, 0)),
            pl.BlockSpec((N_E, E_DIM), lambda i: (0, 0)),
        ],
        out_specs=[
            pl.BlockSpec((TB, 1), lambda i: ((i + NB - 1) % NB, 0)),
            pl.BlockSpec((1, N_E), lambda i: (0, 0)),
            pl.BlockSpec((TB, 1), lambda i: (0, 0)),
        ],
        scratch_shapes=[pltpu.VMEM((TB, N_E), jnp.float32),
                        pltpu.VMEM((TB, N_E), jnp.float32)],
    )(z_flat, cbn)

    zero = jnp.zeros((N_E,), jnp.float32)
    zq, hist = _sc_gather_hist(cb, idx.reshape(TOK), zero)

    inv_n = 1.0 / TOK
    e_mean = (hist[0] + hist[1]) * inv_n
    p = p_sum[0] * inv_n
    rmax_mean = jnp.sum(rm_acc) * inv_n

    commit_loss = (1.0 - rmax_mean) * (1.0 + BETA)
    kl_loss = jnp.sum(p * (jnp.log(p) - math.log(1.0 / N_E)))
    load_balancing_loss = jnp.sum(e_mean * p)
    perplexity = jnp.exp(-jnp.sum(e_mean * jnp.log(e_mean + 1e-6)))
    z_q_st = zq.reshape(z.shape)
    return (z_q_st, commit_loss, kl_loss, load_balancing_loss, cb, perplexity)


# speedup vs baseline: 1.2074x; 1.2074x over previous
import functools
import math

import jax
import jax.numpy as jnp
from jax import lax
from jax.experimental import pallas as pl
from jax.experimental.pallas import tpu as pltpu
from jax.experimental.pallas import tpu_sc as plsc

N_E = 8192
E_DIM = 256
BETA = 0.25
TOK = 9216
TB = 768
NB = TOK // TB
NW = 32
BPW = TOK // NW


def _norm_kernel(w_ref, cb_ref, cbn_ref):
    w = w_ref[...]
    n1 = jnp.sqrt(jnp.sum(w * w, axis=1, keepdims=True))
    cb = w / jnp.maximum(n1, 1e-12)
    n2 = jnp.sqrt(jnp.sum(cb * cb, axis=1, keepdims=True))
    cb_ref[...] = cb
    cbn_ref[...] = (cb / jnp.maximum(n2, 1e-12)).astype(jnp.bfloat16)


def _vq_kernel(z_ref, cbn_ref, idx_ref, p_ref, rm_ref):
    i = pl.program_id(0)
    z = z_ref[...]
    nz = jnp.sqrt(jnp.sum(z * z, axis=1, keepdims=True))
    zn = (z / jnp.maximum(nz, 1e-12)).astype(jnp.bfloat16)
    cbn = cbn_ref[...]
    d = jax.lax.dot_general(zn, cbn, (((1,), (1,)), ((), ())),
                            preferred_element_type=jnp.float32)
    rmax = jnp.max(d, axis=1, keepdims=True)
    iota = jax.lax.broadcasted_iota(jnp.int32, (1, N_E), 1)
    idx = jnp.min(jnp.where(d == rmax, iota, N_E), axis=1, keepdims=True)
    idx_ref[...] = idx

    s = jnp.exp(d - rmax)
    rs = jnp.sum(s, axis=1, keepdims=True)
    pb = jnp.sum(s * (1.0 / rs), axis=0, keepdims=True)

    @pl.when(i == 0)
    def _init():
        p_ref[...] = jnp.zeros_like(p_ref)
        rm_ref[...] = jnp.zeros_like(rm_ref)

    p_ref[...] += pb
    rm_ref[...] += rmax


def _gather_body(cb_hbm, idx_hbm, zero_hbm, zq_hbm, hist_hbm,
                 idx_v, rows_v, ones_v, shared, sem):
    c = lax.axis_index("c")
    s = lax.axis_index("s")
    wid = s * 2 + c
    base = wid * BPW
    pltpu.sync_copy(idx_hbm.at[pl.ds(base, BPW)], idx_v)
    pltpu.async_copy(cb_hbm.at[idx_v], rows_v, sem).wait()
    pltpu.sync_copy(rows_v, zq_hbm.at[pl.ds(base, BPW)])

    def _fill(k, _):
        ones_v[pl.ds(k * 16, 16)] = jnp.ones((16,), jnp.float32)
        return 0
    lax.fori_loop(0, BPW // 16, _fill, 0)

    @pl.when(s == 0)
    def _zero():
        pltpu.sync_copy(zero_hbm, shared)
    plsc.subcore_barrier()
    pltpu.sync_copy(ones_v, shared.at[idx_v], add=True)
    plsc.subcore_barrier()

    @pl.when(s == 0)
    def _write():
        pltpu.sync_copy(shared, hist_hbm.at[c])


def _sc_gather_hist(cb, idx, zero):
    mesh = plsc.VectorSubcoreMesh(core_axis_name="c", subcore_axis_name="s")
    return pl.kernel(
        _gather_body,
        out_type=[
            jax.ShapeDtypeStruct((TOK, E_DIM), jnp.float32),
            jax.ShapeDtypeStruct((2, N_E), jnp.float32),
        ],
        mesh=mesh,
        scratch_types=[
            pltpu.VMEM((BPW,), jnp.int32),
            pltpu.VMEM((BPW, E_DIM), jnp.float32),
            pltpu.VMEM((BPW,), jnp.float32),
            pltpu.VMEM_SHARED((N_E,), jnp.float32),
            pltpu.SemaphoreType.DMA,
        ],
    )(cb, idx, zero)


@functools.partial(jax.jit, static_argnames=())
def kernel(z, W):
    z_flat = z.reshape(-1, E_DIM)

    cb, cbn = pl.pallas_call(
        _norm_kernel,
        out_shape=[
            jax.ShapeDtypeStruct((N_E, E_DIM), jnp.float32),
            jax.ShapeDtypeStruct((N_E, E_DIM), jnp.bfloat16),
        ],
        in_specs=[pl.BlockSpec((N_E, E_DIM), lambda: (0, 0))],
        out_specs=[
            pl.BlockSpec((N_E, E_DIM), lambda: (0, 0)),
            pl.BlockSpec((N_E, E_DIM), lambda: (0, 0)),
        ],
    )(W)

    idx, p_sum, rm_acc = pl.pallas_call(
        _vq_kernel,
        grid=(NB,),
        out_shape=[
            jax.ShapeDtypeStruct((TOK, 1), jnp.int32),
            jax.ShapeDtypeStruct((1, N_E), jnp.float32),
            jax.ShapeDtypeStruct((TB, 1), jnp.float32),
        ],
        in_specs=[
            pl.BlockSpec((TB, E_DIM), lambda i: (i, 0)),
            pl.BlockSpec((N_E, E_DIM), lambda i: (0, 0)),
        ],
        out_specs=[
            pl.BlockSpec((TB, 1), lambda i: (i, 0)),
            pl.BlockSpec((1, N_E), lambda i: (0, 0)),
            pl.BlockSpec((TB, 1), lambda i: (0, 0)),
        ],
    )(z_flat, cbn)

    zero = jnp.zeros((N_E,), jnp.float32)
    zq, hist = _sc_gather_hist(cb, idx.reshape(TOK), zero)

    inv_n = 1.0 / TOK
    e_mean = (hist[0] + hist[1]) * inv_n
    p = p_sum[0] * inv_n
    rmax_mean = jnp.sum(rm_acc) * inv_n

    commit_loss = (1.0 - rmax_mean) * (1.0 + BETA)
    kl_loss = jnp.sum(p * (jnp.log(p) - math.log(1.0 / N_E)))
    load_balancing_loss = jnp.sum(e_mean * p)
    perplexity = jnp.exp(-jnp.sum(e_mean * jnp.log(e_mean + 1e-6)))
    z_q_st = zq.reshape(z.shape)
    return (z_q_st, commit_loss, kl_loss, load_balancing_loss, cb, perplexity)
